# baseline (speedup 1.0000x reference)
import functools
import math

import jax
import jax.numpy as jnp
from jax.experimental import pallas as pl

_BN_INV = 1.0 / math.sqrt(1.0 + 1e-5)


def _mlp_pool_body(x_ref, w_ref, s_ref, t_ref, o_ref, *, K):
    x = x_ref[0]
    h = jnp.dot(x, w_ref[...], preferred_element_type=jnp.float32)
    h = jnp.maximum(h * s_ref[...] + t_ref[...], 0.0)
    SK, O = h.shape
    o_ref[0] = jnp.max(h.reshape(SK // K, K, O), axis=1)


def _mlp_body(x_ref, w_ref, s_ref, t_ref, o_ref):
    h = jnp.dot(x_ref[0], w_ref[...], preferred_element_type=jnp.float32)
    o_ref[0] = jnp.maximum(h * s_ref[...] + t_ref[...], 0.0)


def _head_body(x_ref, w1_ref, b1_ref, gg_ref, gb_ref, w2_ref, b2_ref, o_ref):
    x = x_ref[0]
    h = jnp.dot(x, w1_ref[...], preferred_element_type=jnp.float32) + b1_ref[...]
    m = jnp.mean(h)
    v = jnp.mean((h - m) ** 2)
    h = (h - m) / jnp.sqrt(v + 1e-5)
    h = jnp.maximum(h * gg_ref[...] + gb_ref[...], 0.0)
    o = jnp.dot(h, w2_ref[...], preferred_element_type=jnp.float32) + b2_ref[...]
    mx = jnp.max(o, axis=1, keepdims=True)
    lse = mx + jnp.log(jnp.sum(jnp.exp(o - mx), axis=1, keepdims=True))
    o_ref[0] = o - lse


def _fold_bn(W, b, g, be):
    scale = (g * _BN_INV)[None, :]
    shift = (scale[0] * b + be)[None, :]
    return W.T, scale, shift


def _mlp_pool(x, W, b, g, be, K):
    Bb, SK, C = x.shape
    O = W.shape[0]
    wt, scale, shift = _fold_bn(W, b, g, be)
    return pl.pallas_call(
        functools.partial(_mlp_pool_body, K=K),
        grid=(Bb,),
        in_specs=[
            pl.BlockSpec((1, SK, C), lambda i: (i, 0, 0)),
            pl.BlockSpec((C, O), lambda i: (0, 0)),
            pl.BlockSpec((1, O), lambda i: (0, 0)),
            pl.BlockSpec((1, O), lambda i: (0, 0)),
        ],
        out_specs=pl.BlockSpec((1, SK // K, O), lambda i: (i, 0, 0)),
        out_shape=jax.ShapeDtypeStruct((Bb, SK // K, O), jnp.float32),
    )(x, wt, scale, shift)


def _mlp(x, W, b, g, be):
    Bb, Nn, C = x.shape
    O = W.shape[0]
    wt, scale, shift = _fold_bn(W, b, g, be)
    return pl.pallas_call(
        _mlp_body,
        grid=(Bb,),
        in_specs=[
            pl.BlockSpec((1, Nn, C), lambda i: (i, 0, 0)),
            pl.BlockSpec((C, O), lambda i: (0, 0)),
            pl.BlockSpec((1, O), lambda i: (0, 0)),
            pl.BlockSpec((1, O), lambda i: (0, 0)),
        ],
        out_specs=pl.BlockSpec((1, Nn, O), lambda i: (i, 0, 0)),
        out_shape=jax.ShapeDtypeStruct((Bb, Nn, O), jnp.float32),
    )(x, wt, scale, shift)


def _head(x, params):
    Bb, Nn, C = x.shape
    w1 = params['c1_W'].T
    b1 = params['c1_b'][None, :]
    gg = params['gn_g'][None, :]
    gb = params['gn_b'][None, :]
    w2 = params['c2_W'].T
    b2 = params['c2_b'][None, :]
    O1 = w1.shape[1]
    O2 = w2.shape[1]
    return pl.pallas_call(
        _head_body,
        grid=(Bb,),
        in_specs=[
            pl.BlockSpec((1, Nn, C), lambda i: (i, 0, 0)),
            pl.BlockSpec((C, O1), lambda i: (0, 0)),
            pl.BlockSpec((1, O1), lambda i: (0, 0)),
            pl.BlockSpec((1, O1), lambda i: (0, 0)),
            pl.BlockSpec((1, O1), lambda i: (0, 0)),
            pl.BlockSpec((O1, O2), lambda i: (0, 0)),
            pl.BlockSpec((1, O2), lambda i: (0, 0)),
        ],
        out_specs=pl.BlockSpec((1, Nn, O2), lambda i: (i, 0, 0)),
        out_shape=jax.ShapeDtypeStruct((Bb, Nn, O2), jnp.float32),
    )(x, w1, b1, gg, gb, w2, b2)


def _square_distance(src, dst):
    return (jnp.sum(src ** 2, -1)[:, :, None] + jnp.sum(dst ** 2, -1)[:, None, :]
            - 2.0 * jnp.einsum('bnc,bmc->bnm', src, dst))


def _index_points(points, idx):
    return jax.vmap(lambda p, i: p[i])(points, idx)


def _farthest_point_sample(xyz, npoint):
    Bb, Nn, _ = xyz.shape
    def step(carry, _):
        distance, farthest = carry
        centroid = jnp.take_along_axis(xyz, farthest[:, None, None], axis=1)
        dist = jnp.sum((xyz - centroid) ** 2, -1)
        distance = jnp.minimum(distance, dist)
        new_far = jnp.argmax(distance, -1).astype(jnp.int32)
        return (distance, new_far), farthest
    init = (jnp.full((Bb, Nn), 1e10, dtype=xyz.dtype), jnp.zeros((Bb,), jnp.int32))
    _, cent = jax.lax.scan(step, init, None, length=npoint)
    return jnp.transpose(cent, (1, 0))


def _query_ball_point(radius, nsample, xyz, new_xyz):
    Bb, Nn, _ = xyz.shape
    Ss = new_xyz.shape[1]
    sqrdists = _square_distance(new_xyz, xyz)
    group_idx = jnp.broadcast_to(jnp.arange(Nn, dtype=jnp.int32), (Bb, Ss, Nn))
    group_idx = jnp.where(sqrdists > radius ** 2, Nn, group_idx)
    group_idx = jnp.sort(group_idx, axis=-1)[:, :, :nsample]
    group_first = jnp.broadcast_to(group_idx[:, :, :1], group_idx.shape)
    return jnp.where(group_idx == Nn, group_first, group_idx)


def _sa_layer(xyz_t, pts_t, W, b, g, be, npoint, radius, nsample, group_all):
    Bb, Nn, _ = xyz_t.shape
    if group_all:
        new_xyz = jnp.zeros((Bb, 1, 3), dtype=xyz_t.dtype)
        new_points = jnp.concatenate([xyz_t, pts_t], -1)
        out = _mlp_pool(new_points, W, b, g, be, K=Nn)
    else:
        fps_idx = _farthest_point_sample(xyz_t, npoint)
        new_xyz = _index_points(xyz_t, fps_idx)
        idx = _query_ball_point(radius, nsample, xyz_t, new_xyz)
        grouped_xyz = _index_points(xyz_t, idx) - new_xyz[:, :, None, :]
        grouped_points = _index_points(pts_t, idx)
        new_points = jnp.concatenate([grouped_xyz, grouped_points], -1)
        S, K, C = new_points.shape[1:]
        out = _mlp_pool(new_points.reshape(Bb, S * K, C), W, b, g, be, K=K)
    return new_xyz, out


def _fp_layer(x1, x2, p1, p2, layers):
    Nn = x1.shape[1]
    Ss = x2.shape[1]
    if Ss == 1:
        interpolated = jnp.repeat(p2, Nn, axis=1)
    else:
        dists = _square_distance(x1, x2)
        idx = jnp.argsort(dists, axis=-1)[:, :, :3]
        d3 = jnp.take_along_axis(dists, idx, axis=-1)
        recip = 1.0 / (d3 + 1e-8)
        weight = recip / jnp.sum(recip, axis=2, keepdims=True)
        interpolated = jnp.sum(_index_points(p2, idx) * weight[..., None], axis=2)
    h = interpolated if p1 is None else jnp.concatenate([p1, interpolated], -1)
    for (W, b, g, be) in layers:
        h = _mlp(h, W, b, g, be)
    return h


def _backbone(xyz_t, params):
    l1_xyz, l1_pts = _sa_layer(xyz_t, xyz_t, params['sa1_W'], params['sa1_b'],
                               params['sa1_g'], params['sa1_be'], 512, 0.2, 32, False)
    l2_xyz, l2_pts = _sa_layer(l1_xyz, l1_pts, params['sa2_W'], params['sa2_b'],
                               params['sa2_g'], params['sa2_be'], 128, 0.4, 64, False)
    l3_xyz, l3_pts = _sa_layer(l2_xyz, l2_pts, params['sa3_W'], params['sa3_b'],
                               params['sa3_g'], params['sa3_be'], None, None, None, True)
    l2_pts = _fp_layer(l2_xyz, l3_xyz, l2_pts, l3_pts,
                       [(params['fp3_W'], params['fp3_b'], params['fp3_g'], params['fp3_be'])])
    l1_pts = _fp_layer(l1_xyz, l2_xyz, l1_pts, l2_pts,
                       [(params['fp2_W'], params['fp2_b'], params['fp2_g'], params['fp2_be'])])
    l0_pts = _fp_layer(xyz_t, l1_xyz, None, l1_pts,
                       [(params['fp1a_W'], params['fp1a_b'], params['fp1a_g'], params['fp1a_be']),
                        (params['fp1b_W'], params['fp1b_b'], params['fp1b_g'], params['fp1b_be'])])
    return l0_pts


@jax.jit
def _forward(xyz, xyz_goal, params):
    f0 = _backbone(jnp.transpose(xyz, (0, 2, 1)), params)
    f0g = _backbone(jnp.transpose(xyz_goal, (0, 2, 1)), params)
    x = jnp.concatenate([f0, f0g], axis=-1)
    out = _head(x, params)
    return jnp.transpose(out, (0, 2, 1))


def kernel(xyz, xyz_goal, params):
    return _forward(xyz, xyz_goal, params)
